# Initial kernel scaffold; baseline (speedup 1.0000x reference)
import jax
import jax.numpy as jnp
from jax.experimental import pallas as pl


def kernel(q, k, v):
    raise NotImplementedError("write your pallas kernel here")


# speedup vs baseline: 1.9620x; 1.9620x over previous
import functools
import math

import jax
import jax.numpy as jnp
from jax import lax
from jax.experimental import pallas as pl
from jax.experimental.pallas import tpu as pltpu

S = 2048
H = 16
D = 128
SM_SCALE = 1.0 / math.sqrt(D)
BQ = 512


def _tree_kernel(k_ref, v_ref, kall_ref, vall_ref):
    kc = k_ref[0]
    vc = v_ref[0]
    off = 0
    n = S // 2
    while n >= 1:
        kc2 = kc.reshape(n, 2 * D)
        k0 = kc2[:, :D]
        k1 = kc2[:, D:]
        vc2 = vc.reshape(n, 2 * D)
        v0 = vc2[:, :D]
        v1 = vc2[:, D:]
        kp = 0.5 * (k0 + k1)
        vp_init = 0.5 * (v0 + v1)
        s_self = jnp.sum(kp * kp, axis=1, keepdims=True) * SM_SCALE
        s_c0 = jnp.sum(kp * k0, axis=1, keepdims=True) * SM_SCALE
        s_c1 = jnp.sum(kp * k1, axis=1, keepdims=True) * SM_SCALE
        m = jnp.maximum(s_self, jnp.maximum(s_c0, s_c1))
        e_self = jnp.exp(s_self - m)
        e_c0 = jnp.exp(s_c0 - m)
        e_c1 = jnp.exp(s_c1 - m)
        denom = e_self + e_c0 + e_c1 + 1e-9
        vp = (e_self * vp_init + e_c0 * v0 + e_c1 * v1) / denom
        kall_ref[0, off:off + n, :] = kp
        vall_ref[0, off:off + n, :] = vp
        off += n
        n //= 2
        kc, vc = kp, vp
    kall_ref[0, S - 1:S, :] = jnp.zeros((1, D), jnp.float32)
    vall_ref[0, S - 1:S, :] = jnp.zeros((1, D), jnp.float32)


def _attn_kernel(q_ref, kall_ref, vall_ref, o_ref):
    q = q_ref[0]
    kk = kall_ref[0]
    vv = vall_ref[0]
    s = lax.dot_general(q, kk, (((1,), (1,)), ((), ())),
                        preferred_element_type=jnp.float32) * SM_SCALE
    col = lax.broadcasted_iota(jnp.int32, (BQ, S), 1)
    s = jnp.where(col == S - 1, -1e30, s)
    m = jnp.max(s, axis=1, keepdims=True)
    p = jnp.exp(s - m)
    l = jnp.sum(p, axis=1, keepdims=True)
    o = lax.dot_general(p, vv, (((1,), (0,)), ((), ())),
                        preferred_element_type=jnp.float32)
    o_ref[0] = o / l


@jax.jit
def kernel(q, k, v):
    B, s_, h_, d_ = q.shape
    qT = q[0].transpose(1, 0, 2)
    kT = k[0].transpose(1, 0, 2)
    vT = v[0].transpose(1, 0, 2)

    kall, vall = pl.pallas_call(
        _tree_kernel,
        grid=(H,),
        in_specs=[
            pl.BlockSpec((1, S, D), lambda h: (h, 0, 0)),
            pl.BlockSpec((1, S, D), lambda h: (h, 0, 0)),
        ],
        out_specs=[
            pl.BlockSpec((1, S, D), lambda h: (h, 0, 0)),
            pl.BlockSpec((1, S, D), lambda h: (h, 0, 0)),
        ],
        out_shape=[
            jax.ShapeDtypeStruct((H, S, D), jnp.float32),
            jax.ShapeDtypeStruct((H, S, D), jnp.float32),
        ],
    )(kT, vT)

    out = pl.pallas_call(
        _attn_kernel,
        grid=(H, S // BQ),
        in_specs=[
            pl.BlockSpec((1, BQ, D), lambda h, i: (h, i, 0)),
            pl.BlockSpec((1, S, D), lambda h, i: (h, 0, 0)),
            pl.BlockSpec((1, S, D), lambda h, i: (h, 0, 0)),
        ],
        out_specs=pl.BlockSpec((1, BQ, D), lambda h, i: (h, i, 0)),
        out_shape=jax.ShapeDtypeStruct((H, S, D), jnp.float32),
    )(qT, kall, vall)

    return out.transpose(1, 0, 2)[None]


# speedup vs baseline: 1.9637x; 1.0009x over previous
import functools
import math

import jax
import jax.numpy as jnp
from jax import lax
from jax.experimental import pallas as pl
from jax.experimental.pallas import tpu as pltpu

S = 2048
H = 16
D = 128
SM_SCALE = 1.0 / math.sqrt(D)
BQ = 512


def _tree_kernel(k_ref, v_ref, kall_ref, vall_ref):
    kc = k_ref[0]
    vc = v_ref[0]
    off = 0
    n = S // 2
    while n >= 1:
        kc2 = kc.reshape(n, 2 * D)
        k0 = kc2[:, :D]
        k1 = kc2[:, D:]
        vc2 = vc.reshape(n, 2 * D)
        v0 = vc2[:, :D]
        v1 = vc2[:, D:]
        kp = 0.5 * (k0 + k1)
        vp_init = 0.5 * (v0 + v1)
        s_self = jnp.sum(kp * kp, axis=1, keepdims=True) * SM_SCALE
        s_c0 = jnp.sum(kp * k0, axis=1, keepdims=True) * SM_SCALE
        s_c1 = jnp.sum(kp * k1, axis=1, keepdims=True) * SM_SCALE
        m = jnp.maximum(s_self, jnp.maximum(s_c0, s_c1))
        e_self = jnp.exp(s_self - m)
        e_c0 = jnp.exp(s_c0 - m)
        e_c1 = jnp.exp(s_c1 - m)
        denom = e_self + e_c0 + e_c1 + 1e-9
        vp = (e_self * vp_init + e_c0 * v0 + e_c1 * v1) / denom
        kall_ref[0, off:off + n, :] = kp.astype(jnp.bfloat16)
        vall_ref[0, off:off + n, :] = vp.astype(jnp.bfloat16)
        off += n
        n //= 2
        kc, vc = kp, vp
    kall_ref[0, S - 1:S, :] = jnp.zeros((1, D), jnp.bfloat16)
    vall_ref[0, S - 1:S, :] = jnp.zeros((1, D), jnp.bfloat16)


def _attn_kernel(q_ref, kall_ref, vall_ref, o_ref):
    q = q_ref[0]
    kk = kall_ref[0]
    vv = vall_ref[0]
    s = lax.dot_general(q, kk, (((1,), (1,)), ((), ())),
                        preferred_element_type=jnp.float32) * SM_SCALE
    col = lax.broadcasted_iota(jnp.int32, (BQ, S), 1)
    s = jnp.where(col == S - 1, -1e30, s)
    m = jnp.max(s, axis=1, keepdims=True)
    p = jnp.exp(s - m)
    l = jnp.sum(p, axis=1, keepdims=True)
    o = lax.dot_general(p.astype(jnp.bfloat16), vv, (((1,), (0,)), ((), ())),
                        preferred_element_type=jnp.float32)
    o_ref[0] = o / l


@jax.jit
def kernel(q, k, v):
    B, s_, h_, d_ = q.shape
    qT = q[0].transpose(1, 0, 2).astype(jnp.bfloat16)
    kT = k[0].transpose(1, 0, 2)
    vT = v[0].transpose(1, 0, 2)

    kall, vall = pl.pallas_call(
        _tree_kernel,
        grid=(H,),
        in_specs=[
            pl.BlockSpec((1, S, D), lambda h: (h, 0, 0)),
            pl.BlockSpec((1, S, D), lambda h: (h, 0, 0)),
        ],
        out_specs=[
            pl.BlockSpec((1, S, D), lambda h: (h, 0, 0)),
            pl.BlockSpec((1, S, D), lambda h: (h, 0, 0)),
        ],
        out_shape=[
            jax.ShapeDtypeStruct((H, S, D), jnp.bfloat16),
            jax.ShapeDtypeStruct((H, S, D), jnp.bfloat16),
        ],
    )(kT, vT)

    out = pl.pallas_call(
        _attn_kernel,
        grid=(H, S // BQ),
        in_specs=[
            pl.BlockSpec((1, BQ, D), lambda h, i: (h, i, 0)),
            pl.BlockSpec((1, S, D), lambda h, i: (h, 0, 0)),
            pl.BlockSpec((1, S, D), lambda h, i: (h, 0, 0)),
        ],
        out_specs=pl.BlockSpec((1, BQ, D), lambda h, i: (h, i, 0)),
        out_shape=jax.ShapeDtypeStruct((H, S, D), jnp.float32),
    )(qT, kall, vall)

    return out.transpose(1, 0, 2)[None]
